# baseline (speedup 1.0000x reference)
import functools

import jax
import jax.numpy as jnp
from jax import lax
from jax.experimental import pallas as pl
from jax.experimental.pallas import tpu as pltpu
from jax.experimental.pallas import tpu_sc as plsc

NUM_I = 4096
NUM_J = 200
NUM_INDICES = NUM_I * NUM_J
DIM = 64
LANES = 16
NUM_CORES = 2
NUM_SUBCORES = 16
NUM_WORKERS = NUM_CORES * NUM_SUBCORES
PER_WORKER = NUM_INDICES // NUM_WORKERS
CHUNK = 128
PAIR = 2 * CHUNK
IBLKS = NUM_I // CHUNK
NUM_CHUNKS = PER_WORKER // CHUNK
NUM_SLOTS = NUM_CHUNKS // 2
NGBUF = 3
NTBUF = 2

_mesh = plsc.VectorSubcoreMesh(core_axis_name="c", subcore_axis_name="s")


@functools.partial(
    pl.kernel,
    mesh=_mesh,
    compiler_params=pltpu.CompilerParams(
        use_tc_tiling_on_sc=False, needs_layout_passes=False
    ),
    out_type=jax.ShapeDtypeStruct(
        (NUM_J, DIM // 8, IBLKS, 8, CHUNK), jnp.float32
    ),
    scratch_types=[
        pltpu.VMEM((PER_WORKER,), jnp.int32),
        [pltpu.VMEM((PAIR, DIM), jnp.float32) for _ in range(NGBUF)],
        [pltpu.VMEM((DIM // 8, 8, CHUNK), jnp.float32) for _ in range(NTBUF)],
        [pltpu.SemaphoreType.DMA for _ in range(NGBUF)],
        [pltpu.SemaphoreType.DMA for _ in range(NTBUF)],
    ],
)
def _gather_kernel(idx_hbm, table_hbm, out_hbm, idx_v, rows, trans, gsems, ssems):
    wid = lax.axis_index("s") * NUM_CORES + lax.axis_index("c")
    base = wid * NUM_CHUNKS

    def start_gather(s, gb):
        pltpu.async_copy(
            table_hbm.at[idx_v.at[pl.ds(s * PAIR, PAIR)]], rows[gb], gsems[gb]
        )

    def wait_gather(gb):
        pltpu.make_async_copy(
            table_hbm.at[idx_v.at[pl.ds(0, PAIR)]], rows[gb], gsems[gb]
        ).wait()

    lanes = lax.iota(jnp.int32, LANES)
    cols = [kb * LANES + lanes for kb in range(DIM // LANES)]
    tks = [c >> 3 for c in cols]
    rvs = [c & 7 for c in cols]

    def transpose(gb, h, tb):
        def diag(d, carry):
            off = (lanes + d) & (LANES - 1)
            for mb in range(CHUNK // LANES):
                srcrow = h * CHUNK + mb * LANES + off
                dstrow = mb * LANES + off
                for kb in range(DIM // LANES):
                    vals = plsc.load_gather(rows[gb], [srcrow, cols[kb]])
                    plsc.store_scatter(
                        trans[tb], [tks[kb], rvs[kb], dstrow], vals
                    )
            return carry

        lax.fori_loop(0, LANES, diag, 0)

    def start_store(g, tb):
        gg = base + g
        j = gg // IBLKS
        iblk = gg - j * IBLKS
        pltpu.async_copy(trans[tb], out_hbm.at[j, :, iblk], ssems[tb])

    def wait_store(tb):
        pltpu.make_async_copy(
            trans[tb], out_hbm.at[0, :, 0], ssems[tb]
        ).wait()

    def slot(s, gb, first, last):
        wait_gather(gb)
        for h in range(2):
            g = s * 2 + h
            if not first:
                wait_store(h)
            transpose(gb, h, h)
            start_store(g, h)
        if not last:
            start_gather(s + NGBUF, gb)

    pltpu.sync_copy(idx_hbm.at[pl.ds(wid * PER_WORKER, PER_WORKER)], idx_v)

    for gb in range(NGBUF):
        start_gather(gb, gb)

    slot(0, 0, first=True, last=False)

    def body(t, carry):
        for p in range(3):
            slot(1 + t * 3 + p, (1 + p) % NGBUF, first=False, last=False)
        return carry

    lax.fori_loop(0, (NUM_SLOTS - NGBUF - 1) // 3, body, 0)

    for s in range(NUM_SLOTS - NGBUF, NUM_SLOTS):
        slot(s, s % NGBUF, first=False, last=True)

    for tb in range(NTBUF):
        wait_store(tb)


def kernel(timesteps, table):
    idx = jnp.swapaxes(timesteps, 0, 1).reshape(-1).astype(jnp.int32)
    out5 = _gather_kernel(idx, table)
    return jnp.transpose(out5, (2, 4, 0, 1, 3)).reshape(NUM_I, NUM_J, DIM)


# speedup vs baseline: 1.0001x; 1.0001x over previous
import functools

import jax
import jax.numpy as jnp
from jax import lax
from jax.experimental import pallas as pl
from jax.experimental.pallas import tpu as pltpu
from jax.experimental.pallas import tpu_sc as plsc

NUM_I = 4096
NUM_J = 200
NUM_INDICES = NUM_I * NUM_J
DIM = 64
LANES = 16
NUM_CORES = 2
NUM_SUBCORES = 16
NUM_WORKERS = NUM_CORES * NUM_SUBCORES
PER_WORKER = NUM_INDICES // NUM_WORKERS
CHUNK = 128
PAIR = 2 * CHUNK
IBLKS = NUM_I // CHUNK
NUM_CHUNKS = PER_WORKER // CHUNK
NUM_SLOTS = NUM_CHUNKS // 2
NGBUF = 3
NTBUF = 2

_mesh = plsc.VectorSubcoreMesh(core_axis_name="c", subcore_axis_name="s")


@functools.partial(
    pl.kernel,
    mesh=_mesh,
    compiler_params=pltpu.CompilerParams(
        use_tc_tiling_on_sc=False,
        needs_layout_passes=False,
        disable_bounds_checks=True,
        disable_semaphore_checks=True,
    ),
    out_type=jax.ShapeDtypeStruct(
        (NUM_J, DIM // 8, IBLKS, 8, CHUNK), jnp.float32
    ),
    scratch_types=[
        pltpu.VMEM((PER_WORKER,), jnp.int32),
        [pltpu.VMEM((PAIR, DIM), jnp.float32) for _ in range(NGBUF)],
        [pltpu.VMEM((DIM // 8, 8, CHUNK), jnp.float32) for _ in range(NTBUF)],
        [pltpu.SemaphoreType.DMA for _ in range(NGBUF)],
        [pltpu.SemaphoreType.DMA for _ in range(NTBUF)],
    ],
)
def _gather_kernel(idx_hbm, table_hbm, out_hbm, idx_v, rows, trans, gsems, ssems):
    wid = lax.axis_index("s") * NUM_CORES + lax.axis_index("c")
    base = wid * NUM_CHUNKS

    def start_gather(s, gb):
        pltpu.async_copy(
            table_hbm.at[idx_v.at[pl.ds(s * PAIR, PAIR)]], rows[gb], gsems[gb]
        )

    def wait_gather(gb):
        pltpu.make_async_copy(
            table_hbm.at[idx_v.at[pl.ds(0, PAIR)]], rows[gb], gsems[gb]
        ).wait()

    lanes = lax.iota(jnp.int32, LANES)
    cols = [kb * LANES + lanes for kb in range(DIM // LANES)]
    tks = [c >> 3 for c in cols]
    rvs = [c & 7 for c in cols]

    def transpose(gb, h, tb):
        def diag(d, carry):
            off = (lanes + d) & (LANES - 1)
            for mb in range(CHUNK // LANES):
                srcrow = h * CHUNK + mb * LANES + off
                dstrow = mb * LANES + off
                for kb in range(DIM // LANES):
                    vals = plsc.load_gather(rows[gb], [srcrow, cols[kb]])
                    plsc.store_scatter(
                        trans[tb], [tks[kb], rvs[kb], dstrow], vals
                    )
            return carry

        lax.fori_loop(0, LANES, diag, 0)

    def start_store(g, tb):
        gg = base + g
        j = gg // IBLKS
        iblk = gg - j * IBLKS
        pltpu.async_copy(trans[tb], out_hbm.at[j, :, iblk], ssems[tb])

    def wait_store(tb):
        pltpu.make_async_copy(
            trans[tb], out_hbm.at[0, :, 0], ssems[tb]
        ).wait()

    def slot(s, gb, first, last):
        wait_gather(gb)
        for h in range(2):
            g = s * 2 + h
            if not first:
                wait_store(h)
            transpose(gb, h, h)
            start_store(g, h)
        if not last:
            start_gather(s + NGBUF, gb)

    pltpu.sync_copy(idx_hbm.at[pl.ds(wid * PER_WORKER, PER_WORKER)], idx_v)

    for gb in range(NGBUF):
        start_gather(gb, gb)

    slot(0, 0, first=True, last=False)

    def body(t, carry):
        for p in range(3):
            slot(1 + t * 3 + p, (1 + p) % NGBUF, first=False, last=False)
        return carry

    lax.fori_loop(0, (NUM_SLOTS - NGBUF - 1) // 3, body, 0)

    for s in range(NUM_SLOTS - NGBUF, NUM_SLOTS):
        slot(s, s % NGBUF, first=False, last=True)

    for tb in range(NTBUF):
        wait_store(tb)


def kernel(timesteps, table):
    idx = jnp.swapaxes(timesteps, 0, 1).reshape(-1).astype(jnp.int32)
    out5 = _gather_kernel(idx, table)
    return jnp.transpose(out5, (2, 4, 0, 1, 3)).reshape(NUM_I, NUM_J, DIM)


# speedup vs baseline: 1.0563x; 1.0562x over previous
import functools

import jax
import jax.numpy as jnp
from jax import lax
from jax.experimental import pallas as pl
from jax.experimental.pallas import tpu as pltpu
from jax.experimental.pallas import tpu_sc as plsc

NUM_I = 4096
NUM_J = 200
NUM_INDICES = NUM_I * NUM_J
DIM = 64
PAD_DIM = 128
LANES = 16
NUM_CORES = 2
NUM_SUBCORES = 16
NUM_WORKERS = NUM_CORES * NUM_SUBCORES
PER_WORKER = NUM_INDICES // NUM_WORKERS
CHUNK = 128
IBLKS = NUM_I // CHUNK
NUM_CHUNKS = PER_WORKER // CHUNK
NGBUF = 4
NTBUF = 2

_mesh = plsc.VectorSubcoreMesh(core_axis_name="c", subcore_axis_name="s")


@functools.partial(
    pl.kernel,
    mesh=_mesh,
    compiler_params=pltpu.CompilerParams(needs_layout_passes=False),
    out_type=jax.ShapeDtypeStruct((NUM_J, DIM, NUM_I), jnp.float32),
    scratch_types=[
        pltpu.VMEM((PER_WORKER,), jnp.int32),
        [pltpu.VMEM((CHUNK, PAD_DIM), jnp.float32) for _ in range(NGBUF)],
        [pltpu.VMEM((DIM, CHUNK), jnp.float32) for _ in range(NTBUF)],
        [pltpu.SemaphoreType.DMA for _ in range(NGBUF)],
        [pltpu.SemaphoreType.DMA for _ in range(NTBUF)],
    ],
)
def _gather_kernel(idx_hbm, table_hbm, out_hbm, idx_v, rows, trans, gsems, ssems):
    wid = lax.axis_index("s") * NUM_CORES + lax.axis_index("c")
    base = wid * NUM_CHUNKS

    def start_gather(g, gb):
        pltpu.async_copy(
            table_hbm.at[idx_v.at[pl.ds(g * CHUNK, CHUNK)]], rows[gb], gsems[gb]
        )

    def wait_gather(gb):
        pltpu.make_async_copy(
            table_hbm.at[idx_v.at[pl.ds(0, CHUNK)]], rows[gb], gsems[gb]
        ).wait()

    def transpose(gb, tb):
        lanes = lax.iota(jnp.int32, LANES)

        def diag(d, carry):
            for mb in range(CHUNK // LANES):
                rowv = mb * LANES + ((lanes + d) & (LANES - 1))
                for kb in range(DIM // LANES):
                    colv = kb * LANES + lanes
                    vals = plsc.load_gather(rows[gb], [rowv, colv])
                    plsc.store_scatter(trans[tb], [colv, rowv], vals)
            return carry

        lax.fori_loop(0, LANES, diag, 0)

    def start_store(g, tb):
        gg = base + g
        j = gg // IBLKS
        iblk = gg - j * IBLKS
        pltpu.async_copy(
            trans[tb], out_hbm.at[j, :, pl.ds(iblk * CHUNK, CHUNK)], ssems[tb]
        )

    def wait_store(tb):
        pltpu.make_async_copy(
            trans[tb], out_hbm.at[0, :, pl.ds(0, CHUNK)], ssems[tb]
        ).wait()

    def slot(g, gb, tb, first, last):
        if not first:
            wait_store(tb)
        wait_gather(gb)
        transpose(gb, tb)
        start_store(g, tb)
        if not last:
            start_gather(g + NGBUF, gb)

    pltpu.sync_copy(idx_hbm.at[pl.ds(wid * PER_WORKER, PER_WORKER)], idx_v)

    for gb in range(NGBUF):
        start_gather(gb, gb)

    for g in (0, 1):
        slot(g, g % NGBUF, g % NTBUF, first=True, last=False)

    def body(t, carry):
        for p in range(4):
            g = 2 + t * 4 + p
            slot(g, (2 + p) % NGBUF, p % NTBUF, first=False, last=False)
        return carry

    lax.fori_loop(0, (NUM_CHUNKS - NGBUF - 4) // 4, body, 0)

    for g in range(NUM_CHUNKS - NGBUF - 2, NUM_CHUNKS - NGBUF):
        slot(g, g % NGBUF, g % NTBUF, first=False, last=False)

    for g in range(NUM_CHUNKS - NGBUF, NUM_CHUNKS):
        slot(g, g % NGBUF, g % NTBUF, first=False, last=True)

    for tb in range(NTBUF):
        wait_store(tb)


def kernel(timesteps, table):
    idx = jnp.swapaxes(timesteps, 0, 1).reshape(-1).astype(jnp.int32)
    table_p = jnp.pad(table, ((0, 0), (0, PAD_DIM - DIM)))
    out_p = _gather_kernel(idx, table_p)
    return jnp.transpose(out_p, (2, 0, 1))
